# baseline (speedup 1.0000x reference)
import functools

import jax
import jax.numpy as jnp
from jax import lax
from jax.experimental import pallas as pl
from jax.experimental.pallas import tpu as pltpu
from jax.experimental.pallas import tpu_sc as plsc

EMBED_DIM = 64
NUM_CORES = 2
NUM_SUBCORES = 16
NUM_WORKERS = NUM_CORES * NUM_SUBCORES

RING = 4


def _worker_id():
    return lax.axis_index("s") * NUM_CORES + lax.axis_index("c")


def _make_transpose(vocab: int):
    num_tiles = vocab // 128
    per_w = num_tiles // NUM_WORKERS + 1

    mesh = plsc.VectorSubcoreMesh(core_axis_name="c", subcore_axis_name="s")

    @functools.partial(
        pl.kernel,
        mesh=mesh,
        out_type=jax.ShapeDtypeStruct((vocab // 2, 128), jnp.float32),
        compiler_params=pltpu.CompilerParams(
            use_tc_tiling_on_sc=True, needs_layout_passes=False),
        scratch_types=(
            [pltpu.VMEM((EMBED_DIM, 128), jnp.float32) for _ in range(2)]
            + [pltpu.VMEM((64, 128), jnp.float32) for _ in range(2)]
            + [pltpu.SemaphoreType.DMA for _ in range(4)]
        ),
    )
    def transpose_kernel(tab_hbm, out_hbm, in0, in1, o0, o1, g0, g1, s0, s1):
        wid = _worker_id()
        ins, outs, gsem, ssem = (in0, in1), (o0, o1), (g0, g1), (s0, s1)
        iota = lax.broadcasted_iota(jnp.int32, (16,), 0)

        def g_copy(col0, b):
            return pltpu.make_async_copy(
                tab_hbm.at[:, pl.ds(col0, 128)], ins[b], gsem[b])

        def s_copy(row0, b, rows):
            return pltpu.make_async_copy(
                outs[b].at[pl.ds(0, rows)],
                out_hbm.at[pl.ds(row0, rows)], ssem[b])

        def transpose_block(b, coff, rows):
            def row(r, _):
                for h in range(2):
                    col = coff + 2 * r + h
                    for cb in range(4):
                        v = plsc.load_gather(
                            ins[b], [iota + 16 * cb,
                                     jnp.full((16,), 0, jnp.int32) + col])
                        outs[b][r, pl.ds(h * 64 + 16 * cb, 16)] = v
                return 0
            lax.fori_loop(0, rows, row, 0, unroll=2)

        def do_tile(t, b):
            @pl.when(t < num_tiles)
            def _full():
                g_copy(t * 128, b).start()
                g_copy(t * 128, b).wait()
                transpose_block(b, 0, 64)
                s_copy(t * 64, b, 64).start()
                s_copy(t * 64, b, 64).wait()


        def body(k2, _):
            for b in range(2):
                do_tile(wid + NUM_WORKERS * (2 * k2 + b), b)
            return 0

        lax.fori_loop(0, (per_w + 1) // 2, body, 0)

    return transpose_kernel


def _make_gather(batch: int, seq: int, vocab: int):
    rows_per_w = batch // NUM_WORKERS
    num_blocks = rows_per_w // RING

    mesh = plsc.VectorSubcoreMesh(core_axis_name="c", subcore_axis_name="s")

    @functools.partial(
        pl.kernel,
        mesh=mesh,
        out_type=jax.ShapeDtypeStruct((batch, seq, 2 * EMBED_DIM), jnp.float32),
        compiler_params=pltpu.CompilerParams(use_tc_tiling_on_sc=False),
        scratch_types=(
            [pltpu.VMEM((rows_per_w, seq), jnp.int32)]
            + [pltpu.VMEM((seq, EMBED_DIM), jnp.float32) for _ in range(RING)]
            + [pltpu.SemaphoreType.DMA for _ in range(2 * RING)]
        ),
    )
    def gather_kernel(idx_hbm, table_hbm, out_hbm, idx_v, *rest):
        rows = rest[:RING]
        gsem = rest[RING:2 * RING]
        ssem = rest[2 * RING:]

        wid = _worker_id()
        base = wid * rows_per_w

        pltpu.sync_copy(idx_hbm.at[pl.ds(base, rows_per_w)], idx_v)

        def g_copy(i, r):
            return pltpu.make_async_copy(
                table_hbm.at[idx_v.at[i]], rows[r], gsem[r])

        def s_copy(i, r):
            return pltpu.make_async_copy(
                rows[r], out_hbm.at[base + i, :, pl.ds(0, EMBED_DIM)], ssem[r])

        for r in range(RING):
            g_copy(r, r).start()

        def body(blk, _):
            for r in range(RING):
                i = blk * RING + r
                g_copy(i, r).wait()
                s_copy(i, r).start()
            for r in range(RING):
                i = blk * RING + r
                s_copy(i, r).wait()
                g_copy(i + RING, r).start()
            return 0

        lax.fori_loop(0, num_blocks - 1, body, 0)

        last = (num_blocks - 1) * RING
        for r in range(RING):
            g_copy(last + r, r).wait()
            s_copy(last + r, r).start()
        for r in range(RING):
            s_copy(last + r, r).wait()

    return gather_kernel


def kernel(idx_texts, embed_table):
    batch, seq = idx_texts.shape
    vocab, dim = embed_table.shape
    dense = _make_transpose(vocab)(jnp.swapaxes(embed_table, 0, 1))
    ragged = vocab - (vocab // 128) * 128
    if ragged:
        patch = embed_table[vocab - ragged:].reshape(ragged // 2, 128)
        dense = lax.dynamic_update_slice(
            dense, patch, (vocab // 2 - ragged // 2, 0))
    table = dense.reshape(vocab, dim)
    padded = _make_gather(batch, seq, vocab)(idx_texts, table)
    return padded[:, :, :EMBED_DIM]


# speedup vs baseline: 1.4134x; 1.4134x over previous
import functools

import jax
import jax.numpy as jnp
from jax import lax
from jax.experimental import pallas as pl
from jax.experimental.pallas import tpu as pltpu
from jax.experimental.pallas import tpu_sc as plsc

EMBED_DIM = 64
NUM_CORES = 2
NUM_SUBCORES = 16
NUM_WORKERS = NUM_CORES * NUM_SUBCORES

RING = 4


def _worker_id():
    return lax.axis_index("s") * NUM_CORES + lax.axis_index("c")


def _make_transpose(vocab: int):
    num_tiles = vocab // 128
    per_w = num_tiles // NUM_WORKERS + 1

    mesh = plsc.VectorSubcoreMesh(core_axis_name="c", subcore_axis_name="s")

    @functools.partial(
        pl.kernel,
        mesh=mesh,
        out_type=jax.ShapeDtypeStruct((vocab // 2, 128), jnp.float32),
        compiler_params=pltpu.CompilerParams(
            use_tc_tiling_on_sc=True, needs_layout_passes=False),
        scratch_types=(
            [pltpu.VMEM((EMBED_DIM, 128), jnp.float32) for _ in range(2)]
            + [pltpu.VMEM((64, 128), jnp.float32) for _ in range(2)]
            + [pltpu.SemaphoreType.DMA for _ in range(4)]
        ),
    )
    def transpose_kernel(tab_hbm, out_hbm, in0, in1, o0, o1, g0, g1, s0, s1):
        wid = _worker_id()
        ins, outs, gsem, ssem = (in0, in1), (o0, o1), (g0, g1), (s0, s1)
        iota = lax.broadcasted_iota(jnp.int32, (16,), 0)
        r_pat = iota // 2
        c_pat = (iota % 2) * 64

        def g_copy(t, b):
            return pltpu.make_async_copy(
                tab_hbm.at[:, pl.ds(t * 128, 128)], ins[b], gsem[b])

        def s_copy(t, b):
            return pltpu.make_async_copy(
                outs[b], out_hbm.at[pl.ds(t * 64, 64)], ssem[b])

        def transpose_block(b):
            def row(j, _):
                for cb in range(8):
                    v = ins[b][j, pl.ds(16 * cb, 16)]
                    plsc.store_scatter(
                        outs[b], [r_pat + 8 * cb, c_pat + j], v)
                return 0
            lax.fori_loop(0, EMBED_DIM, row, 0, unroll=4)

        g_copy(wid, 0).start()
        g_copy(wid + NUM_WORKERS, 1).start()

        def body(k2, _):
            for b in range(2):
                t = wid + NUM_WORKERS * (2 * k2 + b)

                @pl.when(t < num_tiles)
                def _work():
                    g_copy(t, b).wait()

                    @pl.when(k2 > 0)
                    def _drain_prev_store():
                        s_copy(t, b).wait()

                    transpose_block(b)

                    @pl.when(t + 2 * NUM_WORKERS < num_tiles)
                    def _prefetch():
                        g_copy(t + 2 * NUM_WORKERS, b).start()

                    s_copy(t, b).start()
            return 0

        lax.fori_loop(0, (per_w + 1) // 2, body, 0)
        for b in range(2):
            s_copy(wid, b).wait()

    return transpose_kernel


def _make_gather(batch: int, seq: int, vocab: int):
    rows_per_w = batch // NUM_WORKERS
    num_blocks = rows_per_w // RING

    mesh = plsc.VectorSubcoreMesh(core_axis_name="c", subcore_axis_name="s")

    @functools.partial(
        pl.kernel,
        mesh=mesh,
        out_type=jax.ShapeDtypeStruct((batch, seq, 2 * EMBED_DIM), jnp.float32),
        compiler_params=pltpu.CompilerParams(use_tc_tiling_on_sc=False),
        scratch_types=(
            [pltpu.VMEM((rows_per_w, seq), jnp.int32)]
            + [pltpu.VMEM((seq, EMBED_DIM), jnp.float32) for _ in range(RING)]
            + [pltpu.SemaphoreType.DMA for _ in range(2 * RING)]
        ),
    )
    def gather_kernel(idx_hbm, table_hbm, out_hbm, idx_v, *rest):
        rows = rest[:RING]
        gsem = rest[RING:2 * RING]
        ssem = rest[2 * RING:]

        wid = _worker_id()
        base = wid * rows_per_w

        pltpu.sync_copy(idx_hbm.at[pl.ds(base, rows_per_w)], idx_v)

        def g_copy(i, r):
            return pltpu.make_async_copy(
                table_hbm.at[idx_v.at[i]], rows[r], gsem[r])

        def s_copy(i, r):
            return pltpu.make_async_copy(
                rows[r], out_hbm.at[base + i, :, pl.ds(0, EMBED_DIM)], ssem[r])

        for r in range(RING):
            g_copy(r, r).start()

        def body(blk, _):
            for r in range(RING):
                i = blk * RING + r
                g_copy(i, r).wait()
                s_copy(i, r).start()
            for r in range(RING):
                i = blk * RING + r
                s_copy(i, r).wait()
                g_copy(i + RING, r).start()
            return 0

        lax.fori_loop(0, num_blocks - 1, body, 0)

        last = (num_blocks - 1) * RING
        for r in range(RING):
            g_copy(last + r, r).wait()
            s_copy(last + r, r).start()
        for r in range(RING):
            s_copy(last + r, r).wait()

    return gather_kernel


def kernel(idx_texts, embed_table):
    batch, seq = idx_texts.shape
    vocab, dim = embed_table.shape
    dense = _make_transpose(vocab)(jnp.swapaxes(embed_table, 0, 1))
    ragged = vocab - (vocab // 128) * 128
    if ragged:
        patch = embed_table[vocab - ragged:].reshape(ragged // 2, 128)
        dense = lax.dynamic_update_slice(
            dense, patch, (vocab // 2 - ragged // 2, 0))
    table = dense.reshape(vocab, dim)
    padded = _make_gather(batch, seq, vocab)(idx_texts, table)
    return padded[:, :, :EMBED_DIM]


# speedup vs baseline: 1.4212x; 1.0055x over previous
import functools

import jax
import jax.numpy as jnp
from jax import lax
from jax.experimental import pallas as pl
from jax.experimental.pallas import tpu as pltpu
from jax.experimental.pallas import tpu_sc as plsc

EMBED_DIM = 64
NUM_CORES = 2
NUM_SUBCORES = 16
NUM_WORKERS = NUM_CORES * NUM_SUBCORES

RING = 4


def _worker_id():
    return lax.axis_index("s") * NUM_CORES + lax.axis_index("c")


def _make_transpose(vocab: int):
    num_tiles = vocab // 128
    per_w = num_tiles // NUM_WORKERS + 1

    mesh = plsc.VectorSubcoreMesh(core_axis_name="c", subcore_axis_name="s")

    @functools.partial(
        pl.kernel,
        mesh=mesh,
        out_type=jax.ShapeDtypeStruct((vocab // 2, 128), jnp.float32),
        compiler_params=pltpu.CompilerParams(
            use_tc_tiling_on_sc=True, needs_layout_passes=False),
        scratch_types=(
            [pltpu.VMEM((EMBED_DIM, 128), jnp.float32) for _ in range(2)]
            + [pltpu.VMEM((64, 128), jnp.float32) for _ in range(2)]
            + [pltpu.SemaphoreType.DMA for _ in range(4)]
        ),
    )
    def transpose_kernel(tab_hbm, out_hbm, in0, in1, o0, o1, g0, g1, s0, s1):
        wid = _worker_id()
        ins, outs, gsem, ssem = (in0, in1), (o0, o1), (g0, g1), (s0, s1)
        iota = lax.broadcasted_iota(jnp.int32, (16,), 0)
        r_pats = [iota // 2 + 8 * cb for cb in range(8)]
        c_pat = (iota % 2) * 64

        def g_copy(t, b):
            return pltpu.make_async_copy(
                tab_hbm.at[:, pl.ds(t * 128, 128)], ins[b], gsem[b])

        def s_copy(t, b):
            return pltpu.make_async_copy(
                outs[b], out_hbm.at[pl.ds(t * 64, 64)], ssem[b])

        def transpose_block(b):
            def row(j, _):
                cj = c_pat + j
                vs = [ins[b][j, pl.ds(16 * cb, 16)] for cb in range(8)]
                for cb in range(8):
                    plsc.store_scatter(outs[b], [r_pats[cb], cj], vs[cb])
                return 0
            lax.fori_loop(0, EMBED_DIM, row, 0, unroll=4)

        g_copy(wid, 0).start()
        g_copy(wid + NUM_WORKERS, 1).start()

        def body(k2, _):
            for b in range(2):
                t = wid + NUM_WORKERS * (2 * k2 + b)

                @pl.when(t < num_tiles)
                def _work():
                    g_copy(t, b).wait()

                    @pl.when(k2 > 0)
                    def _drain_prev_store():
                        s_copy(t, b).wait()

                    transpose_block(b)

                    @pl.when(t + 2 * NUM_WORKERS < num_tiles)
                    def _prefetch():
                        g_copy(t + 2 * NUM_WORKERS, b).start()

                    s_copy(t, b).start()
            return 0

        lax.fori_loop(0, (per_w + 1) // 2, body, 0)
        for b in range(2):
            s_copy(wid, b).wait()

    return transpose_kernel


def _make_gather(batch: int, seq: int, vocab: int):
    rows_per_w = batch // NUM_WORKERS
    num_blocks = rows_per_w // RING

    mesh = plsc.VectorSubcoreMesh(core_axis_name="c", subcore_axis_name="s")

    @functools.partial(
        pl.kernel,
        mesh=mesh,
        out_type=jax.ShapeDtypeStruct((batch, seq, 2 * EMBED_DIM), jnp.float32),
        compiler_params=pltpu.CompilerParams(use_tc_tiling_on_sc=False),
        scratch_types=(
            [pltpu.VMEM((rows_per_w, seq), jnp.int32)]
            + [pltpu.VMEM((seq, EMBED_DIM), jnp.float32) for _ in range(RING)]
            + [pltpu.SemaphoreType.DMA for _ in range(2 * RING)]
        ),
    )
    def gather_kernel(idx_hbm, table_hbm, out_hbm, idx_v, *rest):
        rows = rest[:RING]
        gsem = rest[RING:2 * RING]
        ssem = rest[2 * RING:]

        wid = _worker_id()
        base = wid * rows_per_w

        pltpu.sync_copy(idx_hbm.at[pl.ds(base, rows_per_w)], idx_v)

        def g_copy(i, r):
            return pltpu.make_async_copy(
                table_hbm.at[idx_v.at[i]], rows[r], gsem[r])

        def s_copy(i, r):
            return pltpu.make_async_copy(
                rows[r], out_hbm.at[base + i, :, pl.ds(0, EMBED_DIM)], ssem[r])

        for r in range(RING):
            g_copy(r, r).start()

        def body(blk, _):
            for r in range(RING):
                i = blk * RING + r
                g_copy(i, r).wait()
                s_copy(i, r).start()
            for r in range(RING):
                i = blk * RING + r
                s_copy(i, r).wait()
                g_copy(i + RING, r).start()
            return 0

        lax.fori_loop(0, num_blocks - 1, body, 0)

        last = (num_blocks - 1) * RING
        for r in range(RING):
            g_copy(last + r, r).wait()
            s_copy(last + r, r).start()
        for r in range(RING):
            s_copy(last + r, r).wait()

    return gather_kernel


def kernel(idx_texts, embed_table):
    batch, seq = idx_texts.shape
    vocab, dim = embed_table.shape
    dense = _make_transpose(vocab)(jnp.swapaxes(embed_table, 0, 1))
    ragged = vocab - (vocab // 128) * 128
    if ragged:
        patch = embed_table[vocab - ragged:].reshape(ragged // 2, 128)
        dense = lax.dynamic_update_slice(
            dense, patch, (vocab // 2 - ragged // 2, 0))
    table = dense.reshape(vocab, dim)
    padded = _make_gather(batch, seq, vocab)(idx_texts, table)
    return padded[:, :, :EMBED_DIM]


# speedup vs baseline: 2.3420x; 1.6479x over previous
import functools

import jax
import jax.numpy as jnp
from jax import lax
from jax.experimental import pallas as pl
from jax.experimental.pallas import tpu as pltpu
from jax.experimental.pallas import tpu_sc as plsc

EMBED_DIM = 64
NUM_CORES = 2
NUM_SUBCORES = 16
NUM_WORKERS = NUM_CORES * NUM_SUBCORES

RING = 4


def _worker_id():
    return lax.axis_index("s") * NUM_CORES + lax.axis_index("c")


def _make_gather(batch: int, seq: int, vocab: int):
    rows_per_w = batch // NUM_WORKERS
    num_blocks = rows_per_w // RING

    mesh = plsc.VectorSubcoreMesh(core_axis_name="c", subcore_axis_name="s")

    @functools.partial(
        pl.kernel,
        mesh=mesh,
        out_type=jax.ShapeDtypeStruct((batch, seq, 2 * EMBED_DIM), jnp.float32),
        compiler_params=pltpu.CompilerParams(use_tc_tiling_on_sc=False),
        scratch_types=(
            [pltpu.VMEM((rows_per_w, seq), jnp.int32)]
            + [pltpu.VMEM((seq, EMBED_DIM), jnp.float32) for _ in range(RING)]
            + [pltpu.SemaphoreType.DMA for _ in range(2 * RING)]
        ),
    )
    def gather_kernel(idx_hbm, table_hbm, out_hbm, idx_v, *rest):
        rows = rest[:RING]
        gsem = rest[RING:2 * RING]
        ssem = rest[2 * RING:]

        wid = _worker_id()
        base = wid * rows_per_w

        pltpu.sync_copy(idx_hbm.at[pl.ds(base, rows_per_w)], idx_v)

        def g_copy(i, r):
            return pltpu.make_async_copy(
                table_hbm.at[idx_v.at[i]], rows[r], gsem[r])

        def s_copy(i, r):
            return pltpu.make_async_copy(
                rows[r], out_hbm.at[base + i, :, pl.ds(0, EMBED_DIM)], ssem[r])

        for r in range(RING):
            g_copy(r, r).start()

        def body(blk, _):
            for r in range(RING):
                i = blk * RING + r
                g_copy(i, r).wait()
                s_copy(i, r).start()
            for r in range(RING):
                i = blk * RING + r
                s_copy(i, r).wait()
                g_copy(i + RING, r).start()
            return 0

        lax.fori_loop(0, num_blocks - 1, body, 0)

        last = (num_blocks - 1) * RING
        for r in range(RING):
            g_copy(last + r, r).wait()
            s_copy(last + r, r).start()
        for r in range(RING):
            s_copy(last + r, r).wait()

    return gather_kernel


def kernel(idx_texts, embed_table):
    batch, seq = idx_texts.shape
    vocab, dim = embed_table.shape
    padded = _make_gather(batch, seq, vocab)(idx_texts, embed_table)
    return padded[:, :, :EMBED_DIM]
